# baseline (device time: 18250 ns/iter reference)
import jax
import jax.numpy as jnp
from jax import lax
from jax.experimental import pallas as pl
from jax.experimental.pallas import tpu as pltpu

N_DEV = 4
BLK = 256
DEPTH = 4


def kernel(x):
    m, n = x.shape
    n_blk = m // BLK

    def body(x_ref, out_ref, xbuf, obuf, totals_ref,
             in_sems, out_sems, send_sems, recv_sems):
        my = lax.axis_index("i")

        barrier_sem = pltpu.get_barrier_semaphore()
        for j in range(N_DEV):
            @pl.when(j != my)
            def _():
                pl.semaphore_signal(
                    barrier_sem, inc=1,
                    device_id=(j,), device_id_type=pl.DeviceIdType.MESH,
                )
        pl.semaphore_wait(barrier_sem, N_DEV - 1)

        def load(b):
            return pltpu.make_async_copy(
                x_ref.at[pl.ds(b * BLK, BLK), :],
                xbuf.at[b % DEPTH],
                in_sems.at[b % DEPTH],
            )

        for b in range(min(DEPTH, n_blk)):
            load(b).start()

        row = lax.broadcasted_iota(jnp.int32, (BLK, BLK), 0)
        col = lax.broadcasted_iota(jnp.int32, (BLK, BLK), 1)
        tri = (row >= col).astype(jnp.bfloat16)

        acc = jnp.zeros((1, n), jnp.float32)
        for b in range(n_blk):
            load(b).wait()
            xb = xbuf[b % DEPTH].astype(jnp.bfloat16)
            c = jnp.dot(tri, xb, preferred_element_type=jnp.float32)
            obuf[pl.ds(b * BLK, BLK), :] = (c + acc).astype(obuf.dtype)
            acc = acc + c[BLK - 1 : BLK, :]
            if b + DEPTH < n_blk:
                load(b + DEPTH).start()

        totals_ref[pl.ds(my, 1), :] = acc
        for j in range(N_DEV):
            @pl.when(j != my)
            def _():
                rdma = pltpu.make_async_remote_copy(
                    src_ref=totals_ref.at[pl.ds(my, 1)],
                    dst_ref=totals_ref.at[pl.ds(my, 1)],
                    send_sem=send_sems.at[j],
                    recv_sem=recv_sems.at[my],
                    device_id=(j,),
                    device_id_type=pl.DeviceIdType.MESH,
                )
                rdma.start()

        for j in range(N_DEV):
            @pl.when(j != my)
            def _():
                d = pltpu.make_async_remote_copy(
                    src_ref=totals_ref.at[pl.ds(j, 1)],
                    dst_ref=totals_ref.at[pl.ds(j, 1)],
                    send_sem=send_sems.at[j],
                    recv_sem=recv_sems.at[j],
                    device_id=(j,),
                    device_id_type=pl.DeviceIdType.MESH,
                )
                d.wait_send()
                d.wait_recv()

        slot_ids = lax.broadcasted_iota(jnp.int32, (N_DEV, n), 0)
        offset = jnp.sum(
            jnp.where(slot_ids < my, totals_ref[...], 0.0),
            axis=0,
            keepdims=True,
        )

        def store(b):
            return pltpu.make_async_copy(
                obuf.at[pl.ds(b * BLK, BLK), :],
                out_ref.at[pl.ds(b * BLK, BLK), :],
                out_sems.at[b],
            )

        for b in range(n_blk):
            obuf[pl.ds(b * BLK, BLK), :] = (
                obuf[pl.ds(b * BLK, BLK), :].astype(jnp.float32) + offset
            ).astype(obuf.dtype)
            store(b).start()
        for b in range(n_blk):
            store(b).wait()

    return pl.pallas_call(
        body,
        out_shape=jax.ShapeDtypeStruct((m, n), jnp.bfloat16),
        in_specs=[pl.BlockSpec(memory_space=pl.ANY)],
        out_specs=pl.BlockSpec(memory_space=pl.ANY),
        scratch_shapes=[
            pltpu.VMEM((DEPTH, BLK, n), jnp.float32),
            pltpu.VMEM((m, n), jnp.bfloat16),
            pltpu.VMEM((N_DEV, n), jnp.float32),
            pltpu.SemaphoreType.DMA((DEPTH,)),
            pltpu.SemaphoreType.DMA((n_blk,)),
            pltpu.SemaphoreType.DMA((N_DEV,)),
            pltpu.SemaphoreType.DMA((N_DEV,)),
        ],
        compiler_params=pltpu.CompilerParams(collective_id=0),
    )(x)


# device time: 16038 ns/iter; 1.1379x vs baseline; 1.1379x over previous
import jax
import jax.numpy as jnp
from jax import lax
from jax.experimental import pallas as pl
from jax.experimental.pallas import tpu as pltpu

N_DEV = 4
CHUNKS = (2048, 1792, 256)
SUB = 128
GRP = 512
PREPATCH = 16


def kernel(x):
    m, n = x.shape
    assert sum(CHUNKS) == m
    starts = [sum(CHUNKS[:i]) for i in range(len(CHUNKS))]
    nsub = m // SUB
    ngrp = m // GRP
    spg = GRP // SUB

    def body(x_ref, out_ref, xv0, xv1, xv2, ov, carry, totals_ref,
             in_sems, out_sems, send_sems, recv_sems):
        xvs = [xv0, xv1, xv2]
        my = lax.axis_index("i")

        barrier_sem = pltpu.get_barrier_semaphore()
        for j in range(N_DEV):
            @pl.when(j != my)
            def _():
                pl.semaphore_signal(
                    barrier_sem, inc=1,
                    device_id=(j,), device_id_type=pl.DeviceIdType.MESH,
                )

        def load(cidx):
            return pltpu.make_async_copy(
                x_ref.at[pl.ds(starts[cidx], CHUNKS[cidx]), :],
                xvs[cidx],
                in_sems.at[cidx],
            )

        def store(g):
            return pltpu.make_async_copy(
                ov.at[pl.ds(g * GRP, GRP), :],
                out_ref.at[pl.ds(g * GRP, GRP), :],
                out_sems.at[g],
            )

        def peer_rdma(j, slot):
            return pltpu.make_async_remote_copy(
                src_ref=totals_ref.at[pl.ds(slot, 1)],
                dst_ref=totals_ref.at[pl.ds(slot, 1)],
                send_sem=send_sems.at[j],
                recv_sem=recv_sems.at[slot],
                device_id=(j,),
                device_id_type=pl.DeviceIdType.MESH,
            )

        for cidx in range(len(CHUNKS)):
            load(cidx).start()

        row = lax.broadcasted_iota(jnp.int32, (SUB, SUB), 0)
        col = lax.broadcasted_iota(jnp.int32, (SUB, SUB), 1)
        tri = (row >= col).astype(jnp.bfloat16)

        def do_sub(cidx, s, acc):
            idx = starts[cidx] // SUB + s
            carry[pl.ds(idx, 1), :] = acc
            xb = xvs[cidx][pl.ds(s * SUB, SUB), :].astype(jnp.bfloat16)
            c = jnp.dot(tri, xb, preferred_element_type=jnp.float32)
            ov[pl.ds(idx * SUB, SUB), :] = c.astype(ov.dtype)
            return acc + c[SUB - 1 : SUB, :]

        acc = jnp.zeros((1, n), jnp.float32)
        for cidx in range(len(CHUNKS) - 1):
            load(cidx).wait()
            for s in range(CHUNKS[cidx] // SUB):
                acc = do_sub(cidx, s, acc)

        lastc = len(CHUNKS) - 1
        load(lastc).wait()
        total = acc + jnp.sum(
            xvs[lastc][...], axis=0, keepdims=True, dtype=jnp.float32
        )
        pl.semaphore_wait(barrier_sem, N_DEV - 1)
        totals_ref[pl.ds(my, 1), :] = total
        for j in range(N_DEV):
            @pl.when(j != my)
            def _():
                peer_rdma(j, my).start()

        for s in range(CHUNKS[lastc] // SUB):
            acc = do_sub(lastc, s, acc)
        for idx in range(PREPATCH):
            ov[pl.ds(idx * SUB, SUB), :] = (
                ov[pl.ds(idx * SUB, SUB), :]
                + carry[pl.ds(idx, 1), :].astype(ov.dtype)
            )

        for j in range(N_DEV):
            @pl.when(j != my)
            def _():
                peer_rdma(j, j).wait_recv()

        slot_ids = lax.broadcasted_iota(jnp.int32, (N_DEV, n), 0)
        offset32 = jnp.sum(
            jnp.where(slot_ids < my, totals_ref[...], 0.0),
            axis=0,
            keepdims=True,
        )
        offset = offset32.astype(ov.dtype)

        for g in range(ngrp):
            if (g + 1) * spg <= PREPATCH:
                ov[pl.ds(g * GRP, GRP), :] = ov[pl.ds(g * GRP, GRP), :] + offset
            else:
                for s in range(spg):
                    idx = g * spg + s
                    add = (carry[pl.ds(idx, 1), :] + offset32).astype(ov.dtype)
                    ov[pl.ds(idx * SUB, SUB), :] = (
                        ov[pl.ds(idx * SUB, SUB), :] + add
                    )
            store(g).start()
        for g in range(ngrp):
            store(g).wait()

        for j in range(N_DEV):
            @pl.when(j != my)
            def _():
                peer_rdma(j, my).wait_send()

    return pl.pallas_call(
        body,
        out_shape=jax.ShapeDtypeStruct((m, n), jnp.bfloat16),
        in_specs=[pl.BlockSpec(memory_space=pl.ANY)],
        out_specs=pl.BlockSpec(memory_space=pl.ANY),
        scratch_shapes=[
            pltpu.VMEM((CHUNKS[0], n), jnp.float32),
            pltpu.VMEM((CHUNKS[1], n), jnp.float32),
            pltpu.VMEM((CHUNKS[2], n), jnp.float32),
            pltpu.VMEM((m, n), jnp.bfloat16),
            pltpu.VMEM((nsub, n), jnp.float32),
            pltpu.VMEM((N_DEV, n), jnp.float32),
            pltpu.SemaphoreType.DMA((len(CHUNKS),)),
            pltpu.SemaphoreType.DMA((ngrp,)),
            pltpu.SemaphoreType.DMA((N_DEV,)),
            pltpu.SemaphoreType.DMA((N_DEV,)),
        ],
        compiler_params=pltpu.CompilerParams(collective_id=0),
    )(x)
